# baseline (speedup 1.0000x reference)
import jax
import jax.numpy as jnp
from jax import lax
from jax.experimental import pallas as pl
from jax.experimental.pallas import tpu as pltpu
from jax.experimental.pallas import tpu_sc as plsc

N_ENT = 1000000
RANK = 32
BATCH = 4096

_info = plsc.get_sparse_core_info()
_NC, _NS, _L = _info.num_cores, _info.num_subcores, _info.num_lanes
_NW = _NC * _NS
_BPW = BATCH // _NW
_GROUPS = _BPW // _L


def _body(hq_hbm, rq_hbm, tq_hbm, ent_t, rel_t, bh_t, bt_t,
          pred_out, fh_t_out, fr_t_out, ft_t_out,
          hv, rv, tv, head_v, rel_v, rhs_v, bhv, btv, pred_v,
          gsem, wsem):
    wid = lax.axis_index("s") * _NC + lax.axis_index("c")
    base = wid * _BPW

    pltpu.sync_copy(hq_hbm.at[pl.ds(base, _BPW)], hv)
    pltpu.sync_copy(rq_hbm.at[pl.ds(base, _BPW)], rv)
    pltpu.sync_copy(tq_hbm.at[pl.ds(base, _BPW)], tv)

    copies = []
    for d in range(RANK):
        copies.append(pltpu.async_copy(
            ent_t.at[d].at[hv], head_v.at[pl.ds(d * _BPW, _BPW)], gsem))
        copies.append(pltpu.async_copy(
            rel_t.at[d].at[rv], rel_v.at[pl.ds(d * _BPW, _BPW)], gsem))
        copies.append(pltpu.async_copy(
            ent_t.at[d].at[tv], rhs_v.at[pl.ds(d * _BPW, _BPW)], gsem))
    copies.append(pltpu.async_copy(bh_t.at[0].at[hv], bhv, gsem))
    copies.append(pltpu.async_copy(bt_t.at[0].at[tv], btv, gsem))
    for c in copies:
        c.wait()

    writes = []
    acc = [jnp.zeros((_L,), dtype=jnp.float32) for _ in range(_GROUPS)]
    for d in range(RANK):
        off = d * _BPW
        writes.append(pltpu.async_copy(
            head_v.at[pl.ds(off, _BPW)],
            fh_t_out.at[d].at[pl.ds(base, _BPW)], wsem))
        writes.append(pltpu.async_copy(
            rel_v.at[pl.ds(off, _BPW)],
            fr_t_out.at[d].at[pl.ds(base, _BPW)], wsem))
        writes.append(pltpu.async_copy(
            rhs_v.at[pl.ds(off, _BPW)],
            ft_t_out.at[d].at[pl.ds(base, _BPW)], wsem))
        for g in range(_GROUPS):
            sl = pl.ds(off + g * _L, _L)
            e = (head_v[sl] + rel_v[sl]) - rhs_v[sl]
            acc[g] = acc[g] + e * e
    for g in range(_GROUPS):
        sl = pl.ds(g * _L, _L)
        pred_v[sl] = (bhv[sl] + btv[sl]) - acc[g]

    pltpu.sync_copy(pred_v, pred_out.at[pl.ds(base, _BPW)])
    for w in writes:
        w.wait()


@jax.jit
def kernel(queries, entity, rel, bh, bt):
    mesh = plsc.VectorSubcoreMesh(core_axis_name="c", subcore_axis_name="s")
    f32 = jnp.float32
    run = pl.kernel(
        _body,
        mesh=mesh,
        compiler_params=pltpu.CompilerParams(use_tc_tiling_on_sc=False),
        out_type=[
            jax.ShapeDtypeStruct((BATCH,), f32),
            jax.ShapeDtypeStruct((RANK, BATCH), f32),
            jax.ShapeDtypeStruct((RANK, BATCH), f32),
            jax.ShapeDtypeStruct((RANK, BATCH), f32),
        ],
        scratch_types=[
            pltpu.VMEM((_BPW,), jnp.int32),
            pltpu.VMEM((_BPW,), jnp.int32),
            pltpu.VMEM((_BPW,), jnp.int32),
            pltpu.VMEM((_BPW * RANK,), f32),
            pltpu.VMEM((_BPW * RANK,), f32),
            pltpu.VMEM((_BPW * RANK,), f32),
            pltpu.VMEM((_BPW,), f32),
            pltpu.VMEM((_BPW,), f32),
            pltpu.VMEM((_BPW,), f32),
            pltpu.SemaphoreType.DMA,
            pltpu.SemaphoreType.DMA,
        ],
    )
    queries = queries.astype(jnp.int32)
    hq = queries[:, 0]
    rq = queries[:, 1]
    tq = queries[:, 2]
    pred, fh_t, fr_t, ft_t = run(hq, rq, tq, entity.T, rel.T,
                                 bh.T, bt.T)
    return (pred.reshape(BATCH, 1), fh_t.T, fr_t.T, ft_t.T)


# speedup vs baseline: 1.0364x; 1.0364x over previous
import jax
import jax.numpy as jnp
from jax import lax
from jax.experimental import pallas as pl
from jax.experimental.pallas import tpu as pltpu
from jax.experimental.pallas import tpu_sc as plsc

RANK = 32
BATCH = 4096

_info = plsc.get_sparse_core_info()
_NC, _NS, _L = _info.num_cores, _info.num_subcores, _info.num_lanes
_NW = _NC * _NS
_BPW = BATCH // _NW
_GROUPS = _BPW // _L


def _body(hq_hbm, rq_hbm, tq_hbm, ent_t, rel_t, bh_t, bt_t,
          pred_out, fh_t_out, fr_t_out, ft_t_out,
          hv, rv, tv, head_v, rel_v, rhs_v, bhv, btv, pred_v,
          gsem, wsem):
    wid = lax.axis_index("s") * _NC + lax.axis_index("c")
    base = wid * _BPW

    pltpu.sync_copy(hq_hbm.at[pl.ds(base, _BPW)], hv)
    pltpu.sync_copy(rq_hbm.at[pl.ds(base, _BPW)], rv)
    pltpu.sync_copy(tq_hbm.at[pl.ds(base, _BPW)], tv)

    copies = []
    for d in range(RANK):
        copies.append(pltpu.async_copy(
            ent_t.at[d].at[hv], head_v.at[pl.ds(d * _BPW, _BPW)], gsem))
        copies.append(pltpu.async_copy(
            rel_t.at[d].at[rv], rel_v.at[pl.ds(d * _BPW, _BPW)], gsem))
        copies.append(pltpu.async_copy(
            ent_t.at[d].at[tv], rhs_v.at[pl.ds(d * _BPW, _BPW)], gsem))
    copies.append(pltpu.async_copy(bh_t.at[hv], bhv, gsem))
    copies.append(pltpu.async_copy(bt_t.at[tv], btv, gsem))
    for c in copies:
        c.wait()

    writes = []
    acc = [jnp.zeros((_L,), dtype=jnp.float32) for _ in range(_GROUPS)]
    for d in range(RANK):
        off = d * _BPW
        writes.append(pltpu.async_copy(
            head_v.at[pl.ds(off, _BPW)],
            fh_t_out.at[d].at[pl.ds(base, _BPW)], wsem))
        writes.append(pltpu.async_copy(
            rel_v.at[pl.ds(off, _BPW)],
            fr_t_out.at[d].at[pl.ds(base, _BPW)], wsem))
        writes.append(pltpu.async_copy(
            rhs_v.at[pl.ds(off, _BPW)],
            ft_t_out.at[d].at[pl.ds(base, _BPW)], wsem))
        for g in range(_GROUPS):
            sl = pl.ds(off + g * _L, _L)
            e = (head_v[sl] + rel_v[sl]) - rhs_v[sl]
            acc[g] = acc[g] + e * e
    for g in range(_GROUPS):
        sl = pl.ds(g * _L, _L)
        pred_v[sl] = (bhv[sl] + btv[sl]) - acc[g]

    pltpu.sync_copy(pred_v, pred_out.at[pl.ds(base, _BPW)])
    for w in writes:
        w.wait()


@jax.jit
def kernel(queries, entity, rel, bh, bt):
    mesh = plsc.VectorSubcoreMesh(core_axis_name="c", subcore_axis_name="s")
    f32 = jnp.float32
    run = pl.kernel(
        _body,
        mesh=mesh,
        compiler_params=pltpu.CompilerParams(use_tc_tiling_on_sc=False),
        out_type=[
            jax.ShapeDtypeStruct((BATCH,), f32),
            jax.ShapeDtypeStruct((RANK, BATCH), f32),
            jax.ShapeDtypeStruct((RANK, BATCH), f32),
            jax.ShapeDtypeStruct((RANK, BATCH), f32),
        ],
        scratch_types=[
            pltpu.VMEM((_BPW,), jnp.int32),
            pltpu.VMEM((_BPW,), jnp.int32),
            pltpu.VMEM((_BPW,), jnp.int32),
            pltpu.VMEM((_BPW * RANK,), f32),
            pltpu.VMEM((_BPW * RANK,), f32),
            pltpu.VMEM((_BPW * RANK,), f32),
            pltpu.VMEM((_BPW,), f32),
            pltpu.VMEM((_BPW,), f32),
            pltpu.VMEM((_BPW,), f32),
            pltpu.SemaphoreType.DMA,
            pltpu.SemaphoreType.DMA,
        ],
    )
    queries = queries.astype(jnp.int32)
    hq = queries[:, 0]
    rq = queries[:, 1]
    tq = queries[:, 2]
    pred, fh_t, fr_t, ft_t = run(hq, rq, tq, entity.T, rel.T,
                                 bh.reshape(-1), bt.reshape(-1))
    return (pred.reshape(BATCH, 1), fh_t.T, fr_t.T, ft_t.T)


# speedup vs baseline: 3.7352x; 3.6041x over previous
import jax
import jax.numpy as jnp
from jax import lax
from jax.experimental import pallas as pl
from jax.experimental.pallas import tpu as pltpu
from jax.experimental.pallas import tpu_sc as plsc

RANK = 32
BATCH = 4096

_info = plsc.get_sparse_core_info()
_NC, _NS, _L = _info.num_cores, _info.num_subcores, _info.num_lanes
_NW = _NC * _NS
_BPW = BATCH // _NW
_GROUPS = _BPW // _L


def _body(hq_hbm, rq_hbm, tq_hbm, entf, relf, bh, bt,
          pred_out, fh_out, fr_out, ft_out,
          hv, rv, tv, hD, rD, tD, hI, rI, tI, oI, bhv, btv, pred_v,
          gsem, wsem):
    wid = lax.axis_index("s") * _NC + lax.axis_index("c")
    base = wid * _BPW

    pltpu.sync_copy(hq_hbm.at[pl.ds(base, _BPW)], hv)
    pltpu.sync_copy(rq_hbm.at[pl.ds(base, _BPW)], rv)
    pltpu.sync_copy(tq_hbm.at[pl.ds(base, _BPW)], tv)
    q32 = lax.iota(jnp.int32, _L) * jnp.int32(RANK)
    for src, dst in ((hv, hI), (rv, rI), (tv, tI)):
        for g in range(_GROUPS):
            v32 = src[pl.ds(g * _L, _L)] * jnp.int32(RANK)
            for d in range(RANK):
                dst[pl.ds(d * _BPW + g * _L, _L)] = v32 + jnp.int32(d)
    obase = base * RANK
    for g in range(_GROUPS):
        ov = q32 + (obase + g * _L * RANK)
        for d in range(RANK):
            oI[pl.ds(d * _BPW + g * _L, _L)] = ov + jnp.int32(d)

    copies = [
        pltpu.async_copy(entf.at[hI], hD, gsem),
        pltpu.async_copy(relf.at[rI], rD, gsem),
        pltpu.async_copy(entf.at[tI], tD, gsem),
        pltpu.async_copy(bh.at[hv], bhv, gsem),
        pltpu.async_copy(bt.at[tv], btv, gsem),
    ]
    for c in copies:
        c.wait()

    writes = [
        pltpu.async_copy(hD, fh_out.at[oI], wsem),
        pltpu.async_copy(rD, fr_out.at[oI], wsem),
        pltpu.async_copy(tD, ft_out.at[oI], wsem),
    ]
    acc = [jnp.zeros((_L,), dtype=jnp.float32) for _ in range(_GROUPS)]
    for d in range(RANK):
        off = d * _BPW
        for g in range(_GROUPS):
            sl = pl.ds(off + g * _L, _L)
            e = (hD[sl] + rD[sl]) - tD[sl]
            acc[g] = acc[g] + e * e
    for g in range(_GROUPS):
        sl = pl.ds(g * _L, _L)
        pred_v[sl] = (bhv[sl] + btv[sl]) - acc[g]

    pltpu.sync_copy(pred_v, pred_out.at[pl.ds(base, _BPW)])
    for w in writes:
        w.wait()


@jax.jit
def kernel(queries, entity, rel, bh, bt):
    mesh = plsc.VectorSubcoreMesh(core_axis_name="c", subcore_axis_name="s")
    f32 = jnp.float32
    i32 = jnp.int32
    run = pl.kernel(
        _body,
        mesh=mesh,
        compiler_params=pltpu.CompilerParams(use_tc_tiling_on_sc=False),
        out_type=[
            jax.ShapeDtypeStruct((BATCH,), f32),
            jax.ShapeDtypeStruct((BATCH * RANK,), f32),
            jax.ShapeDtypeStruct((BATCH * RANK,), f32),
            jax.ShapeDtypeStruct((BATCH * RANK,), f32),
        ],
        scratch_types=[
            pltpu.VMEM((_BPW,), i32),
            pltpu.VMEM((_BPW,), i32),
            pltpu.VMEM((_BPW,), i32),
            pltpu.VMEM((_BPW * RANK,), f32),
            pltpu.VMEM((_BPW * RANK,), f32),
            pltpu.VMEM((_BPW * RANK,), f32),
            pltpu.VMEM((_BPW * RANK,), i32),
            pltpu.VMEM((_BPW * RANK,), i32),
            pltpu.VMEM((_BPW * RANK,), i32),
            pltpu.VMEM((_BPW * RANK,), i32),
            pltpu.VMEM((_BPW,), f32),
            pltpu.VMEM((_BPW,), f32),
            pltpu.VMEM((_BPW,), f32),
            pltpu.SemaphoreType.DMA,
            pltpu.SemaphoreType.DMA,
        ],
    )
    queries = queries.astype(i32)
    hq = queries[:, 0]
    rq = queries[:, 1]
    tq = queries[:, 2]
    pred, fh, fr, ft = run(hq, rq, tq,
                           entity.reshape(-1), rel.reshape(-1),
                           bh.reshape(-1), bt.reshape(-1))
    return (pred.reshape(BATCH, 1), fh.reshape(BATCH, RANK),
            fr.reshape(BATCH, RANK), ft.reshape(BATCH, RANK))


# speedup vs baseline: 5.8289x; 1.5606x over previous
import jax
import jax.numpy as jnp
from jax import lax
from jax.experimental import pallas as pl
from jax.experimental.pallas import tpu as pltpu
from jax.experimental.pallas import tpu_sc as plsc

RANK = 32
BATCH = 4096

_info = plsc.get_sparse_core_info()
_NC, _NS, _L = _info.num_cores, _info.num_subcores, _info.num_lanes
_NW = _NC * _NS
_BPW = BATCH // _NW
_GROUPS = _BPW // _L


def _body(hq_hbm, rq_hbm, tq_hbm, entf, relf, bh, bt,
          pred_out, fh_out, fr_out, ft_out,
          hv, rv, tv, hD, rD, tD, hI, rI, tI, bhv, btv, pred_v,
          gsem, wsem):
    wid = lax.axis_index("s") * _NC + lax.axis_index("c")
    base = wid * _BPW

    pltpu.sync_copy(hq_hbm.at[pl.ds(base, _BPW)], hv)
    pltpu.sync_copy(rq_hbm.at[pl.ds(base, _BPW)], rv)
    pltpu.sync_copy(tq_hbm.at[pl.ds(base, _BPW)], tv)
    for src, dst in ((hv, hI), (rv, rI), (tv, tI)):
        for g in range(_GROUPS):
            v32 = src[pl.ds(g * _L, _L)] * jnp.int32(RANK)
            for d in range(RANK):
                dst[pl.ds(d * _BPW + g * _L, _L)] = v32 + jnp.int32(d)
    copies = [
        pltpu.async_copy(bh.at[hv], bhv, gsem),
        pltpu.async_copy(bt.at[tv], btv, gsem),
    ]
    for d in range(RANK):
        sl = pl.ds(d * _BPW, _BPW)
        copies.append(pltpu.async_copy(entf.at[hI.at[sl]], hD.at[sl], gsem))
        copies.append(pltpu.async_copy(relf.at[rI.at[sl]], rD.at[sl], gsem))
        copies.append(pltpu.async_copy(entf.at[tI.at[sl]], tD.at[sl], gsem))
    for c in copies:
        c.wait()

    writes = []
    acc = [jnp.zeros((_L,), dtype=jnp.float32) for _ in range(_GROUPS)]
    for d in range(RANK):
        off = d * _BPW
        writes.append(pltpu.async_copy(
            hD.at[pl.ds(off, _BPW)],
            fh_out.at[d].at[pl.ds(base, _BPW)], wsem))
        writes.append(pltpu.async_copy(
            rD.at[pl.ds(off, _BPW)],
            fr_out.at[d].at[pl.ds(base, _BPW)], wsem))
        writes.append(pltpu.async_copy(
            tD.at[pl.ds(off, _BPW)],
            ft_out.at[d].at[pl.ds(base, _BPW)], wsem))
        for g in range(_GROUPS):
            sl = pl.ds(off + g * _L, _L)
            e = (hD[sl] + rD[sl]) - tD[sl]
            acc[g] = acc[g] + e * e
    for g in range(_GROUPS):
        sl = pl.ds(g * _L, _L)
        pred_v[sl] = (bhv[sl] + btv[sl]) - acc[g]

    pltpu.sync_copy(pred_v, pred_out.at[pl.ds(base, _BPW)])
    for w in writes:
        w.wait()


@jax.jit
def kernel(queries, entity, rel, bh, bt):
    mesh = plsc.VectorSubcoreMesh(core_axis_name="c", subcore_axis_name="s")
    f32 = jnp.float32
    i32 = jnp.int32
    run = pl.kernel(
        _body,
        mesh=mesh,
        compiler_params=pltpu.CompilerParams(use_tc_tiling_on_sc=False),
        out_type=[
            jax.ShapeDtypeStruct((BATCH,), f32),
            jax.ShapeDtypeStruct((RANK, BATCH), f32),
            jax.ShapeDtypeStruct((RANK, BATCH), f32),
            jax.ShapeDtypeStruct((RANK, BATCH), f32),
        ],
        scratch_types=[
            pltpu.VMEM((_BPW,), i32),
            pltpu.VMEM((_BPW,), i32),
            pltpu.VMEM((_BPW,), i32),
            pltpu.VMEM((_BPW * RANK,), f32),
            pltpu.VMEM((_BPW * RANK,), f32),
            pltpu.VMEM((_BPW * RANK,), f32),
            pltpu.VMEM((_BPW * RANK,), i32),
            pltpu.VMEM((_BPW * RANK,), i32),
            pltpu.VMEM((_BPW * RANK,), i32),
            pltpu.VMEM((_BPW,), f32),
            pltpu.VMEM((_BPW,), f32),
            pltpu.VMEM((_BPW,), f32),
            pltpu.SemaphoreType.DMA,
            pltpu.SemaphoreType.DMA,
        ],
    )
    queries = queries.astype(i32)
    hq = queries[:, 0]
    rq = queries[:, 1]
    tq = queries[:, 2]
    pred, fh, fr, ft = run(hq, rq, tq,
                           entity.reshape(-1), rel.reshape(-1),
                           bh.reshape(-1), bt.reshape(-1))
    return (pred.reshape(BATCH, 1), fh.T, fr.T, ft.T)
